# baseline (speedup 1.0000x reference)
import jax
import jax.numpy as jnp
from jax import lax
from jax.experimental import pallas as pl
from jax.experimental.pallas import tpu as pltpu
from jax.experimental.pallas import tpu_sc as plsc

N_ATOMS = 1638400
N_MOL = 16384
W_FORCE = 0.999
W_ENERGY = 0.001

NC = 2
NS = 16
LANES = 16

NW = NC * NS
APW = N_ATOMS // NW
APL = APW // LANES
ROUNDS = 8
CPL = APL // ROUNDS
SEG_F = 3 * CPL
MPT = N_MOL // NS


def _sc_body(p_hbm, t_hbm, idx_hbm, out_hbm,
             p_buf, t_buf, i_buf, acc, red, tmp, slab, sem):
    c = lax.axis_index("c")
    s = lax.axis_index("s")
    wid = c * NS + s
    wbase = wid * APW

    lanes = lax.iota(jnp.int32, LANES)
    zero16 = jnp.zeros((LANES,), jnp.float32)

    def _zero(j, carry):
        acc[pl.ds(j * LANES, LANES)] = zero16
        return carry
    lax.fori_loop(0, N_MOL // LANES, _zero, 0)

    def _load_round(r):
        descs = []
        for L in range(LANES):
            a0 = wbase + L * APL + r * CPL
            descs.append(pltpu.async_copy(
                p_hbm.at[pl.ds(3 * a0, SEG_F)],
                p_buf.at[pl.ds(L * SEG_F, SEG_F)], sem))
            descs.append(pltpu.async_copy(
                t_hbm.at[pl.ds(3 * a0, SEG_F)],
                t_buf.at[pl.ds(L * SEG_F, SEG_F)], sem))
            descs.append(pltpu.async_copy(
                idx_hbm.at[pl.ds(a0, CPL)],
                i_buf.at[pl.ds(L * CPL, CPL)], sem))
        for d in descs:
            d.wait()

    lane_i = lanes * CPL
    lane_f = lanes * SEG_F

    def _step(i, carry):
        acc_v, mprev = carry
        col = jnp.full((LANES,), i, jnp.int32)
        m = plsc.load_gather(i_buf, [lane_i + col])
        c0 = lane_f + col * 3
        p0 = plsc.load_gather(p_buf, [c0])
        t0 = plsc.load_gather(t_buf, [c0])
        p1 = plsc.load_gather(p_buf, [c0 + 1])
        t1 = plsc.load_gather(t_buf, [c0 + 1])
        p2 = plsc.load_gather(p_buf, [c0 + 2])
        t2 = plsc.load_gather(t_buf, [c0 + 2])
        d0 = p0 - t0
        d1 = p1 - t1
        d2 = p2 - t2
        e = d0 * d0 + d1 * d1 + d2 * d2
        changed = m != mprev
        plsc.addupdate_scatter(acc, [mprev], acc_v, mask=changed)
        acc_v = jnp.where(changed, e, acc_v + e)
        return acc_v, m

    acc_v = zero16
    mprev = None
    for r in range(ROUNDS):
        _load_round(r)
        if r == 0:
            mprev = plsc.load_gather(i_buf, [lane_i])
        acc_v, mprev = lax.fori_loop(0, CPL, _step, (acc_v, mprev))

    for k in range(LANES):
        plsc.addupdate_scatter(acc, [mprev], acc_v, mask=(lanes == k))

    pltpu.sync_copy(acc, slab.at[pl.ds(s * N_MOL, N_MOL)])
    plsc.subcore_barrier()
    pltpu.sync_copy(slab.at[pl.ds(s * MPT, MPT)], red)
    for j in range(1, NS):
        pltpu.sync_copy(slab.at[pl.ds(j * N_MOL + s * MPT, MPT)], tmp)

        def _acc(q, carry):
            sl = pl.ds(q * LANES, LANES)
            red[sl] = red[sl] + tmp[sl]
            return carry
        lax.fori_loop(0, MPT // LANES, _acc, 0)
    pltpu.sync_copy(red, out_hbm.at[c, pl.ds(s * MPT, MPT)])


_sc_partial = pl.kernel(
    _sc_body,
    out_type=jax.ShapeDtypeStruct((NC, N_MOL), jnp.float32),
    mesh=plsc.VectorSubcoreMesh(core_axis_name="c", subcore_axis_name="s"),
    scratch_types=[
        pltpu.VMEM((LANES * SEG_F,), jnp.float32),
        pltpu.VMEM((LANES * SEG_F,), jnp.float32),
        pltpu.VMEM((LANES * CPL,), jnp.int32),
        pltpu.VMEM((N_MOL,), jnp.float32),
        pltpu.VMEM((MPT,), jnp.float32),
        pltpu.VMEM((MPT,), jnp.float32),
        pltpu.VMEM_SHARED((NS * N_MOL,), jnp.float32),
        pltpu.SemaphoreType.DMA,
    ],
    compiler_params=pltpu.CompilerParams(needs_layout_passes=False),
)


def _fin_body(pm0, pm1, cnt, ep, et, tot, lf, le):
    counts = cnt[...].astype(jnp.float32)
    force = (pm0[...] + pm1[...]) / (3.0 * counts)
    d = ep[...] - et[...]
    energy = (d * d) / counts
    tot[...] = W_FORCE * force + W_ENERGY * energy
    lf[...] = force
    le[...] = energy


_R = 128


def kernel(per_atom_force_predict, per_atom_force_true,
           per_molecule_energy_predict, per_molecule_energy_true,
           atomic_subsystem_indices, atomic_subsystem_counts):
    p_flat = per_atom_force_predict.reshape(-1)
    t_flat = per_atom_force_true.reshape(-1)
    partial = _sc_partial(p_flat, t_flat, atomic_subsystem_indices)

    shp = jax.ShapeDtypeStruct((_R, N_MOL // _R), jnp.float32)
    tot, lf, le = pl.pallas_call(
        _fin_body,
        out_shape=(shp, shp, shp),
    )(
        partial[0].reshape(_R, -1),
        partial[1].reshape(_R, -1),
        atomic_subsystem_counts.reshape(_R, -1),
        per_molecule_energy_predict.reshape(_R, -1),
        per_molecule_energy_true.reshape(_R, -1),
    )
    out = (tot.reshape(N_MOL, 1), lf.reshape(N_MOL, 1), le.reshape(N_MOL, 1))
    return out


# speedup vs baseline: 4.0611x; 4.0611x over previous
import jax
import jax.numpy as jnp
from jax import lax
from jax.experimental import pallas as pl
from jax.experimental.pallas import tpu as pltpu
from jax.experimental.pallas import tpu_sc as plsc

N_ATOMS = 1638400
N_MOL = 16384
W_FORCE = 0.999
W_ENERGY = 0.001

NC = 2
NS = 16
LANES = 16

NW = NC * NS
APW = N_ATOMS // NW
APL = APW // LANES
ROUNDS = 4
CPL = APL // ROUNDS
LC = LANES * CPL
MPT = N_MOL // NS


def _sc_body(p_hbm, t_hbm, idx_hbm, out_hbm,
             p_buf, t_buf, i_buf, acc, red, tmp, slab, sem):
    c = lax.axis_index("c")
    s = lax.axis_index("s")
    wid = c * NS + s
    wbase = wid * APW

    lanes = lax.iota(jnp.int32, LANES)
    zero16 = jnp.zeros((LANES,), jnp.float32)

    def _zero(j, carry):
        acc[pl.ds(j * LANES, LANES)] = zero16
        return carry
    lax.fori_loop(0, N_MOL // LANES, _zero, 0)

    lane_base = lanes * CPL
    pos_x = lane_base
    pos_y = lane_base + LC
    pos_z = lane_base + 2 * LC

    def _step(i, carry):
        acc_v, mprev = carry
        m = plsc.load_gather(i_buf, [lane_base + i])
        dx = plsc.load_gather(p_buf, [pos_x + i]) - \
            plsc.load_gather(t_buf, [pos_x + i])
        dy = plsc.load_gather(p_buf, [pos_y + i]) - \
            plsc.load_gather(t_buf, [pos_y + i])
        dz = plsc.load_gather(p_buf, [pos_z + i]) - \
            plsc.load_gather(t_buf, [pos_z + i])
        e = dx * dx + dy * dy + dz * dz
        changed = m != mprev
        plsc.addupdate_scatter(acc, [mprev], acc_v, mask=changed)
        acc_v = jnp.where(changed, e, acc_v + e)
        return acc_v, m

    acc_v = zero16
    mprev = None
    for r in range(ROUNDS):
        descs = []
        for L in range(LANES):
            a0 = wbase + L * APL + r * CPL
            for k in range(3):
                descs.append(pltpu.async_copy(
                    p_hbm.at[pl.ds(k * N_ATOMS + a0, CPL)],
                    p_buf.at[pl.ds(k * LC + L * CPL, CPL)], sem))
                descs.append(pltpu.async_copy(
                    t_hbm.at[pl.ds(k * N_ATOMS + a0, CPL)],
                    t_buf.at[pl.ds(k * LC + L * CPL, CPL)], sem))
            descs.append(pltpu.async_copy(
                idx_hbm.at[pl.ds(a0, CPL)],
                i_buf.at[pl.ds(L * CPL, CPL)], sem))
        for d in descs:
            d.wait()
        if r == 0:
            mprev = plsc.load_gather(i_buf, [lane_base])
        acc_v, mprev = lax.fori_loop(0, CPL, _step, (acc_v, mprev))

    for k in range(LANES):
        plsc.addupdate_scatter(acc, [mprev], acc_v, mask=(lanes == k))

    pltpu.sync_copy(acc, slab.at[pl.ds(s * N_MOL, N_MOL)])
    plsc.subcore_barrier()
    pltpu.sync_copy(slab.at[pl.ds(s * MPT, MPT)], red)
    for j in range(1, NS):
        pltpu.sync_copy(slab.at[pl.ds(j * N_MOL + s * MPT, MPT)], tmp)

        def _acc(q, carry):
            sl = pl.ds(q * LANES, LANES)
            red[sl] = red[sl] + tmp[sl]
            return carry
        lax.fori_loop(0, MPT // LANES, _acc, 0)
    pltpu.sync_copy(red, out_hbm.at[c, pl.ds(s * MPT, MPT)])


_sc_partial = pl.kernel(
    _sc_body,
    out_type=jax.ShapeDtypeStruct((NC, N_MOL), jnp.float32),
    mesh=plsc.VectorSubcoreMesh(core_axis_name="c", subcore_axis_name="s"),
    scratch_types=[
        pltpu.VMEM((3 * LC,), jnp.float32),
        pltpu.VMEM((3 * LC,), jnp.float32),
        pltpu.VMEM((LC,), jnp.int32),
        pltpu.VMEM((N_MOL,), jnp.float32),
        pltpu.VMEM((MPT,), jnp.float32),
        pltpu.VMEM((MPT,), jnp.float32),
        pltpu.VMEM_SHARED((NS * N_MOL,), jnp.float32),
        pltpu.SemaphoreType.DMA,
    ],
    compiler_params=pltpu.CompilerParams(needs_layout_passes=False),
)


def _fin_body(pm0, pm1, cnt, ep, et, tot, lf, le):
    counts = cnt[...].astype(jnp.float32)
    force = (pm0[...] + pm1[...]) / (3.0 * counts)
    d = ep[...] - et[...]
    energy = (d * d) / counts
    tot[...] = W_FORCE * force + W_ENERGY * energy
    lf[...] = force
    le[...] = energy


_R = 128


def kernel(per_atom_force_predict, per_atom_force_true,
           per_molecule_energy_predict, per_molecule_energy_true,
           atomic_subsystem_indices, atomic_subsystem_counts):
    p_planes = per_atom_force_predict.T.reshape(-1)
    t_planes = per_atom_force_true.T.reshape(-1)
    partial = _sc_partial(p_planes, t_planes, atomic_subsystem_indices)

    shp = jax.ShapeDtypeStruct((_R, N_MOL // _R), jnp.float32)
    tot, lf, le = pl.pallas_call(
        _fin_body,
        out_shape=(shp, shp, shp),
    )(
        partial[0].reshape(_R, -1),
        partial[1].reshape(_R, -1),
        atomic_subsystem_counts.reshape(_R, -1),
        per_molecule_energy_predict.reshape(_R, -1),
        per_molecule_energy_true.reshape(_R, -1),
    )
    out = (tot.reshape(N_MOL, 1), lf.reshape(N_MOL, 1), le.reshape(N_MOL, 1))
    return out


# speedup vs baseline: 7.0909x; 1.7461x over previous
import jax
import jax.numpy as jnp
from jax import lax
from jax.experimental import pallas as pl
from jax.experimental.pallas import tpu as pltpu
from jax.experimental.pallas import tpu_sc as plsc

N_ATOMS = 1638400
N_MOL = 16384
W_FORCE = 0.999
W_ENERGY = 0.001

NC = 2
NS = 16
LANES = 16

NW = NC * NS
APW = N_ATOMS // NW
APL = APW // LANES
ROUNDS = 4
CPL = APL // ROUNDS
LC = LANES * CPL
MPT = N_MOL // NS


def _sc_body(d_hbm, idx_hbm, out_hbm,
             d_buf, i_buf, acc, red, tmp, slab, sem):
    c = lax.axis_index("c")
    s = lax.axis_index("s")
    wid = c * NS + s
    wbase = wid * APW

    lanes = lax.iota(jnp.int32, LANES)
    zero16 = jnp.zeros((LANES,), jnp.float32)

    def _zero(j, carry):
        acc[pl.ds(j * LANES, LANES)] = zero16
        return carry
    lax.fori_loop(0, N_MOL // LANES, _zero, 0)

    lane_base = lanes * CPL
    pos_x = lane_base
    pos_y = lane_base + LC
    pos_z = lane_base + 2 * LC

    def _step(i, carry):
        acc_v, mprev = carry
        m = plsc.load_gather(i_buf, [lane_base + i])
        dx = plsc.load_gather(d_buf, [pos_x + i])
        dy = plsc.load_gather(d_buf, [pos_y + i])
        dz = plsc.load_gather(d_buf, [pos_z + i])
        e = dx * dx + dy * dy + dz * dz
        changed = m != mprev
        plsc.addupdate_scatter(acc, [mprev], acc_v, mask=changed)
        acc_v = jnp.where(changed, e, acc_v + e)
        return acc_v, m

    acc_v = zero16
    mprev = None
    for r in range(ROUNDS):
        descs = []
        for L in range(LANES):
            a0 = wbase + L * APL + r * CPL
            for k in range(3):
                descs.append(pltpu.async_copy(
                    d_hbm.at[pl.ds(k * N_ATOMS + a0, CPL)],
                    d_buf.at[pl.ds(k * LC + L * CPL, CPL)], sem))
            descs.append(pltpu.async_copy(
                idx_hbm.at[pl.ds(a0, CPL)],
                i_buf.at[pl.ds(L * CPL, CPL)], sem))
        for d in descs:
            d.wait()
        if r == 0:
            mprev = plsc.load_gather(i_buf, [lane_base])
        acc_v, mprev = lax.fori_loop(0, CPL, _step, (acc_v, mprev),
                                     unroll=8)

    for k in range(LANES):
        plsc.addupdate_scatter(acc, [mprev], acc_v, mask=(lanes == k))

    pltpu.sync_copy(acc, slab.at[pl.ds(s * N_MOL, N_MOL)])
    plsc.subcore_barrier()
    pltpu.sync_copy(slab.at[pl.ds(s * MPT, MPT)], red)
    for j in range(1, NS):
        pltpu.sync_copy(slab.at[pl.ds(j * N_MOL + s * MPT, MPT)], tmp)

        def _acc(q, carry):
            sl = pl.ds(q * LANES, LANES)
            red[sl] = red[sl] + tmp[sl]
            return carry
        lax.fori_loop(0, MPT // LANES, _acc, 0)
    pltpu.sync_copy(red, out_hbm.at[c, pl.ds(s * MPT, MPT)])


_sc_partial = pl.kernel(
    _sc_body,
    out_type=jax.ShapeDtypeStruct((NC, N_MOL), jnp.float32),
    mesh=plsc.VectorSubcoreMesh(core_axis_name="c", subcore_axis_name="s"),
    scratch_types=[
        pltpu.VMEM((3 * LC,), jnp.float32),
        pltpu.VMEM((LC,), jnp.int32),
        pltpu.VMEM((N_MOL,), jnp.float32),
        pltpu.VMEM((MPT,), jnp.float32),
        pltpu.VMEM((MPT,), jnp.float32),
        pltpu.VMEM_SHARED((NS * N_MOL,), jnp.float32),
        pltpu.SemaphoreType.DMA,
    ],
    compiler_params=pltpu.CompilerParams(needs_layout_passes=False),
)


def _fin_body(pm0, pm1, cnt, ep, et, tot, lf, le):
    counts = cnt[...].astype(jnp.float32)
    force = (pm0[...] + pm1[...]) / (3.0 * counts)
    d = ep[...] - et[...]
    energy = (d * d) / counts
    tot[...] = W_FORCE * force + W_ENERGY * energy
    lf[...] = force
    le[...] = energy


_R = 128


def kernel(per_atom_force_predict, per_atom_force_true,
           per_molecule_energy_predict, per_molecule_energy_true,
           atomic_subsystem_indices, atomic_subsystem_counts):
    d_planes = (per_atom_force_predict - per_atom_force_true).T.reshape(-1)
    partial = _sc_partial(d_planes, atomic_subsystem_indices)

    shp = jax.ShapeDtypeStruct((_R, N_MOL // _R), jnp.float32)
    tot, lf, le = pl.pallas_call(
        _fin_body,
        out_shape=(shp, shp, shp),
    )(
        partial[0].reshape(_R, -1),
        partial[1].reshape(_R, -1),
        atomic_subsystem_counts.reshape(_R, -1),
        per_molecule_energy_predict.reshape(_R, -1),
        per_molecule_energy_true.reshape(_R, -1),
    )
    out = (tot.reshape(N_MOL, 1), lf.reshape(N_MOL, 1), le.reshape(N_MOL, 1))
    return out


# speedup vs baseline: 20.2740x; 2.8592x over previous
import jax
import jax.numpy as jnp
from jax import lax
from jax.experimental import pallas as pl
from jax.experimental.pallas import tpu as pltpu
from jax.experimental.pallas import tpu_sc as plsc

N_ATOMS = 1638400
N_MOL = 16384
W_FORCE = 0.999
W_ENERGY = 0.001

NC = 2
NS = 16
LANES = 16

NW = NC * NS
APW = N_ATOMS // NW
APL = APW // LANES
ROUNDS = 5
CPL = APL // ROUNDS
GPR = CPL // 128
MPT = N_MOL // NS


def _sc_body(d_hbm, idx_hbm, out_hbm,
             d_buf, i_buf, acc, red, tmp, slab, sem):
    c = lax.axis_index("c")
    s = lax.axis_index("s")
    wid = c * NS + s
    wbase = wid * APW

    lanes = lax.iota(jnp.int32, LANES)
    zero16 = jnp.zeros((LANES,), jnp.float32)

    def _zero(j, carry):
        acc[pl.ds(j * LANES, LANES)] = zero16
        return carry
    lax.fori_loop(0, N_MOL // LANES, _zero, 0)

    lane_d = lanes * (3 * CPL)
    lane_i = lanes * CPL

    acc_v = zero16
    mprev = None
    for r in range(ROUNDS):
        descs = []
        for L in range(LANES):
            a0 = wbase + L * APL + r * CPL
            descs.append(pltpu.async_copy(
                d_hbm.at[pl.ds(3 * a0, 3 * CPL)],
                d_buf.at[pl.ds(L * 3 * CPL, 3 * CPL)], sem))
            descs.append(pltpu.async_copy(
                idx_hbm.at[pl.ds(a0, CPL)],
                i_buf.at[pl.ds(L * CPL, CPL)], sem))
        for d in descs:
            d.wait()
        if r == 0:
            mprev = plsc.load_gather(i_buf, [lane_i])
        for g in range(GPR):
            def _body(l, carry, _g=g):
                acc_v, mprev = carry
                m = plsc.load_gather(i_buf, [lane_i + (_g * 128 + l)])
                base = lane_d + (_g * 384 + l)
                dx = plsc.load_gather(d_buf, [base])
                dy = plsc.load_gather(d_buf, [base + 128])
                dz = plsc.load_gather(d_buf, [base + 256])
                e = dx * dx + dy * dy + dz * dz
                changed = m != mprev
                plsc.addupdate_scatter(acc, [mprev], acc_v, mask=changed)
                acc_v = jnp.where(changed, e, acc_v + e)
                return acc_v, m
            acc_v, mprev = plsc.parallel_loop(
                0, 128, unroll=8, carry=(acc_v, mprev))(_body)

    for k in range(LANES):
        plsc.addupdate_scatter(acc, [mprev], acc_v, mask=(lanes == k))

    pltpu.sync_copy(acc, slab.at[pl.ds(s * N_MOL, N_MOL)])
    plsc.subcore_barrier()
    pltpu.sync_copy(slab.at[pl.ds(s * MPT, MPT)], red)
    for j in range(1, NS):
        pltpu.sync_copy(slab.at[pl.ds(j * N_MOL + s * MPT, MPT)], tmp)

        def _acc(q, carry):
            sl = pl.ds(q * LANES, LANES)
            red[sl] = red[sl] + tmp[sl]
            return carry
        lax.fori_loop(0, MPT // LANES, _acc, 0)
    pltpu.sync_copy(red, out_hbm.at[c, pl.ds(s * MPT, MPT)])


_sc_partial = pl.kernel(
    _sc_body,
    out_type=jax.ShapeDtypeStruct((NC, N_MOL), jnp.float32),
    mesh=plsc.VectorSubcoreMesh(core_axis_name="c", subcore_axis_name="s"),
    scratch_types=[
        pltpu.VMEM((LANES * 3 * CPL,), jnp.float32),
        pltpu.VMEM((LANES * CPL,), jnp.int32),
        pltpu.VMEM((N_MOL,), jnp.float32),
        pltpu.VMEM((MPT,), jnp.float32),
        pltpu.VMEM((MPT,), jnp.float32),
        pltpu.VMEM_SHARED((NS * N_MOL,), jnp.float32),
        pltpu.SemaphoreType.DMA,
    ],
    compiler_params=pltpu.CompilerParams(needs_layout_passes=False),
)


def _fin_body(pm0, pm1, cnt, ep, et, tot, lf, le):
    counts = cnt[...].astype(jnp.float32)
    force = (pm0[...] + pm1[...]) / (3.0 * counts)
    d = ep[...] - et[...]
    energy = (d * d) / counts
    tot[...] = W_FORCE * force + W_ENERGY * energy
    lf[...] = force
    le[...] = energy


_R = 128


def kernel(per_atom_force_predict, per_atom_force_true,
           per_molecule_energy_predict, per_molecule_energy_true,
           atomic_subsystem_indices, atomic_subsystem_counts):
    d_flat = ((per_atom_force_predict - per_atom_force_true)
              .T.reshape(3, N_ATOMS // 128, 128)
              .transpose(1, 0, 2).reshape(-1))
    partial = _sc_partial(d_flat, atomic_subsystem_indices)

    shp = jax.ShapeDtypeStruct((_R, N_MOL // _R), jnp.float32)
    tot, lf, le = pl.pallas_call(
        _fin_body,
        out_shape=(shp, shp, shp),
    )(
        partial[0].reshape(_R, -1),
        partial[1].reshape(_R, -1),
        atomic_subsystem_counts.reshape(_R, -1),
        per_molecule_energy_predict.reshape(_R, -1),
        per_molecule_energy_true.reshape(_R, -1),
    )
    out = (tot.reshape(N_MOL, 1), lf.reshape(N_MOL, 1), le.reshape(N_MOL, 1))
    return out
